# baseline (speedup 1.0000x reference)
import numpy as np
import jax
import jax.numpy as jnp
from jax import lax
from jax.experimental import pallas as pl
from jax.experimental.pallas import tpu as pltpu
from jax.experimental.pallas import tpu_sc as plsc

N = 10000
E = 320000
D = 128
GRID_SIZE = 5
SPLINE_ORDER = 3
COEF = GRID_SIZE + SPLINE_ORDER

NSUB = 16
EPW = E // NSUB
CHUNK = 128
NFULL = EPW // CHUNK
TAIL = EPW - NFULL * CHUNK
SUP_C = 12
SUP_P = SUP_C // 2
SUP_E = SUP_C * CHUNK

C3 = 80
NCH3 = EPW // C3
NTRI = (NCH3 - 1) // 3
SUP3_C = 24
SUP3_E = SUP3_C * C3
EPAD = 321536
RPS = 624
RPS_LAST = N - 15 * RPS

ROWB = 1000
NBLK = N // ROWB

_G = np.arange(-SPLINE_ORDER, GRID_SIZE + SPLINE_ORDER + 1,
               dtype=np.float32) * np.float32(2.0 / GRID_SIZE) - np.float32(1.0)


_BSCALE = float(1.0 / (6.0 * (np.float64(_G[1]) - np.float64(_G[0])) ** 3))


def _bsplines(x):
    g = _G
    s = [x - g[j] for j in range(len(g))]
    bs = [jnp.logical_and(x >= g[j], x < g[j + 1]).astype(x.dtype)
          for j in range(len(g) - 1)]
    for k in range(1, SPLINE_ORDER + 1):
        bs = [s[j] * bs[j] - s[j + k + 1] * bs[j + 1]
              for j in range(len(bs) - 1)]
    return bs


def _dot(a, b):
    return lax.dot_general(a, b, (((1,), (0,)), ((), ())),
                           precision=lax.Precision.DEFAULT,
                           preferred_element_type=jnp.float32)


def _dinv(deg_ref):
    d = deg_ref[...][:, 0:1]
    return jnp.where(d > 0.0, lax.rsqrt(d), 0.0)


def _kan3(h, hb, hs, sb, ss, ib, iw):
    sil = jax.nn.silu(h)
    b = jnp.concatenate(_bsplines(h), axis=1)
    zh = _dot(sil, hb[...]) + _dot(b, hs[...])
    zs = _dot(sil, sb[...]) + _dot(b, ss[...])
    zi = _dot(sil, ib[...]) + _dot(b, iw[...])
    return zh, zs, zi


_ROWSPEC = pl.BlockSpec((ROWB, D), lambda i: (i, 0))
_DEGSPEC = pl.BlockSpec((ROWB, 16), lambda i: (i, 0))
_WSPEC = pl.BlockSpec((D, D), lambda i: (0, 0))
_SWSPEC = pl.BlockSpec((COEF * D, D), lambda i: (0, 0))
_OUT = jax.ShapeDtypeStruct((N, D), jnp.float32)


def _tc_layer0(x, degu, degd, ws):
    def body(x_ref, du_ref, dd_ref, hb, hs, sb, ss, ib, iw,
             zh_ref, ks_ref, ki_ref):
        dinvu = _dinv(du_ref)
        dinvd = _dinv(dd_ref)
        zh, zs, zi = _kan3(x_ref[...], hb, hs, sb, ss, ib, iw)
        zh_ref[...] = zh
        ks_ref[...] = zs * dinvu
        ki_ref[...] = zi * dinvd
    return pl.pallas_call(
        body, grid=(NBLK,),
        in_specs=[_ROWSPEC, _DEGSPEC, _DEGSPEC,
                  _WSPEC, _SWSPEC, _WSPEC, _SWSPEC, _WSPEC, _SWSPEC],
        out_specs=[_ROWSPEC, _ROWSPEC, _ROWSPEC],
        out_shape=[_OUT, _OUT, _OUT],
    )(x, degu, degd, *ws)


def _tc_layer1(zh0, au, ad, degu, degd, ws):
    def body(zh0_ref, au_ref, ad_ref, du_ref, dd_ref,
             hb, hs, sb, ss, ib, iw, zh_ref, ks_ref, ki_ref):
        dinvu = _dinv(du_ref)
        dinvd = _dinv(dd_ref)
        h = jax.nn.relu(zh0_ref[...] + dinvu * au_ref[...] + dinvd * ad_ref[...])
        zh, zs, zi = _kan3(h, hb, hs, sb, ss, ib, iw)
        zh_ref[...] = zh
        ks_ref[...] = zs * dinvu
        ki_ref[...] = zi * dinvd
    return pl.pallas_call(
        body, grid=(NBLK,),
        in_specs=[_ROWSPEC, _ROWSPEC, _ROWSPEC, _DEGSPEC, _DEGSPEC,
                  _WSPEC, _SWSPEC, _WSPEC, _SWSPEC, _WSPEC, _SWSPEC],
        out_specs=[_ROWSPEC, _ROWSPEC, _ROWSPEC],
        out_shape=[_OUT, _OUT, _OUT],
    )(zh0, au, ad, degu, degd, *ws)


def _tc_final(zh1, au, ad, degu, degd):
    def body(zh_ref, au_ref, ad_ref, du_ref, dd_ref, o_ref):
        dinvu = _dinv(du_ref)
        dinvd = _dinv(dd_ref)
        o_ref[...] = jax.nn.relu(
            zh_ref[...] + dinvu * au_ref[...] + dinvd * ad_ref[...])
    return pl.pallas_call(
        body, grid=(NBLK,),
        in_specs=[_ROWSPEC, _ROWSPEC, _ROWSPEC, _DEGSPEC, _DEGSPEC],
        out_specs=_ROWSPEC,
        out_shape=_OUT,
    )(zh1, au, ad, degu, degd)


def _sc_mesh():
    return plsc.VectorSubcoreMesh(core_axis_name="c", subcore_axis_name="s")


def _rowcopy(sid, src, dst):
    @pl.when(sid < 15)
    def _():
        off = pl.multiple_of(sid * RPS, 8)
        pltpu.sync_copy(src.at[pl.ds(off, RPS)], dst.at[pl.ds(off, RPS)])

    @pl.when(sid == 15)
    def _():
        pltpu.sync_copy(src.at[pl.ds(15 * RPS, RPS_LAST)],
                        dst.at[pl.ds(15 * RPS, RPS_LAST)])


DEGW = 16


def _sc_deg(cu, cd, zerosW, onesW):
    def body(cu_hbm, cd_hbm, z_hbm, o_hbm, du_hbm, dd_hbm,
             acc, cidx, cidxT, ones_v, call_v):
        cid = lax.axis_index("c")
        sid = lax.axis_index("s")
        _rowcopy(sid, z_hbm, acc)
        pltpu.sync_copy(o_hbm, ones_v)
        plsc.subcore_barrier()
        base = sid * EPW

        def run(col_hbm):
            pltpu.sync_copy(col_hbm.at[pl.ds(base, EPW)], call_v)

            def step(j, carry):
                for i in range(CHUNK // 16):
                    cidx[pl.ds(16 * i, 16)] = call_v[pl.ds(j * CHUNK + 16 * i, 16)]
                pltpu.sync_copy(ones_v, acc.at[cidx], add=True)
                return carry
            lax.fori_loop(0, NFULL, step, 0)
            for i in range(TAIL // 16):
                cidxT[pl.ds(16 * i, 16)] = call_v[pl.ds(NFULL * CHUNK + 16 * i, 16)]
            pltpu.sync_copy(ones_v.at[pl.ds(0, TAIL)], acc.at[cidxT], add=True)

        @pl.when(cid == 0)
        def _():
            run(cu_hbm)

        @pl.when(cid == 1)
        def _():
            run(cd_hbm)

        plsc.subcore_barrier()

        @pl.when(cid == 0)
        def _():
            _rowcopy(sid, acc, du_hbm)

        @pl.when(cid == 1)
        def _():
            _rowcopy(sid, acc, dd_hbm)

    f = pl.kernel(
        body,
        out_type=(jax.ShapeDtypeStruct((N, DEGW), jnp.float32),
                  jax.ShapeDtypeStruct((N, DEGW), jnp.float32)),
        mesh=_sc_mesh(),
        compiler_params=pltpu.CompilerParams(use_tc_tiling_on_sc=False),
        scratch_types=(
            pltpu.VMEM_SHARED((N, DEGW), jnp.float32),
            pltpu.VMEM((CHUNK,), jnp.int32),
            pltpu.VMEM((TAIL,), jnp.int32),
            pltpu.VMEM((CHUNK, DEGW), jnp.float32),
            pltpu.VMEM((EPW,), jnp.int32),
        ),
    )
    return f(cu, cd, zerosW, onesW)


def _sc_prop(ks, ki, ru, cu, rd, cd, zerosD):
    def body(ks_hbm, ki_hbm, ru_hbm, cu_hbm, rd_hbm, cd_hbm, z_hbm,
             au_hbm, ad_hbm, acc,
             ridxs, cidxs, rows, rall_v, call_v, gsems, ssems):
        cid = lax.axis_index("c")
        sid = lax.axis_index("s")
        _rowcopy(sid, z_hbm, acc)
        plsc.subcore_barrier()
        base = sid * EPW

        def run(tab_hbm, row_hbm, col_hbm):
            def stage(s):
                pltpu.sync_copy(row_hbm.at[pl.ds(base + s * SUP3_E, SUP3_E)], rall_v)
                pltpu.sync_copy(col_hbm.at[pl.ds(base + s * SUP3_E, SUP3_E)], call_v)

            def load_idx(jj, s):
                loc = jj * C3 - (jj // SUP3_C) * SUP3_E
                for i in range(C3 // 16):
                    ridxs[s][pl.ds(16 * i, 16)] = rall_v[pl.ds(loc + 16 * i, 16)]
                    cidxs[s][pl.ds(16 * i, 16)] = call_v[pl.ds(loc + 16 * i, 16)]

            def start_gather(s):
                pltpu.async_copy(tab_hbm.at[ridxs[s]], rows[s], gsems[s])

            def wait_gather(s):
                pltpu.make_async_copy(tab_hbm.at[ridxs[s]], rows[s], gsems[s]).wait()

            def start_scatter(s):
                pltpu.async_copy(rows[s], acc.at[cidxs[s]], ssems[s], add=True)

            def wait_scatter(s):
                pltpu.make_async_copy(rows[s], acc.at[cidxs[s]], ssems[s]).wait()

            stage(0)
            for s in range(3):
                load_idx(s, s)
                start_gather(s)

            def triple(t, carry):
                for s in range(3):
                    wait_gather(s)
                    start_scatter(s)

                @pl.when(jnp.logical_and(t % (SUP3_C // 3) == SUP3_C // 3 - 1,
                                         t < NTRI - 1))
                def _():
                    stage((t + 1) // (SUP3_C // 3))

                for s in range(3):
                    jj = 3 * t + 3 + s

                    @pl.when(jj <= NCH3 - 1)
                    def _():
                        wait_scatter(s)
                        load_idx(jj, s)
                        start_gather(s)
                return carry
            lax.fori_loop(0, NTRI, triple, 0)

            wait_gather(0)
            start_scatter(0)
            wait_scatter(0)
            wait_scatter(1)
            wait_scatter(2)

        @pl.when(cid == 0)
        def _():
            run(ks_hbm, ru_hbm, cu_hbm)

        @pl.when(cid == 1)
        def _():
            run(ki_hbm, rd_hbm, cd_hbm)

        plsc.subcore_barrier()

        @pl.when(cid == 0)
        def _():
            _rowcopy(sid, acc, au_hbm)

        @pl.when(cid == 1)
        def _():
            _rowcopy(sid, acc, ad_hbm)

    f = pl.kernel(
        body,
        out_type=(jax.ShapeDtypeStruct((N, D), jnp.float32),
                  jax.ShapeDtypeStruct((N, D), jnp.float32)),
        mesh=_sc_mesh(),
        scratch_types=(
            pltpu.VMEM_SHARED((N, D), jnp.float32),
            [pltpu.VMEM((C3,), jnp.int32)] * 3,
            [pltpu.VMEM((C3,), jnp.int32)] * 3,
            [pltpu.VMEM((C3, D), jnp.float32)] * 3,
            pltpu.VMEM((SUP3_E,), jnp.int32),
            pltpu.VMEM((SUP3_E,), jnp.int32),
            [pltpu.SemaphoreType.DMA] * 3,
            [pltpu.SemaphoreType.DMA] * 3,
        ),
    )
    return f(ks, ki, ru, cu, rd, cd, zerosD)


def _prep_ws(bw, sw):
    sw_t = jnp.transpose(sw, (2, 1, 0)).reshape(COEF * D, D)
    return bw.T, sw_t * jnp.float32(_BSCALE)


def kernel(x, Ld, Lu, l0_har_base_w, l0_har_spline_w, l0_sol_base_w,
           l0_sol_spline_w, l0_irr_base_w, l0_irr_spline_w, l1_har_base_w,
           l1_har_spline_w, l1_sol_base_w, l1_sol_spline_w, l1_irr_base_w,
           l1_irr_spline_w):
    epad = jnp.zeros((EPAD - E,), jnp.int32)
    ru, cu = (jnp.concatenate([Lu[0], epad]), jnp.concatenate([Lu[1], epad]))
    rd, cd = (jnp.concatenate([Ld[0], epad]), jnp.concatenate([Ld[1], epad]))
    zerosD = jnp.zeros((N, D), jnp.float32)
    zerosW = jnp.zeros((N, DEGW), jnp.float32)
    onesW = jnp.ones((CHUNK, DEGW), jnp.float32)

    hb0, hs0 = _prep_ws(l0_har_base_w, l0_har_spline_w)
    sb0, ss0 = _prep_ws(l0_sol_base_w, l0_sol_spline_w)
    ib0, iw0 = _prep_ws(l0_irr_base_w, l0_irr_spline_w)
    hb1, hs1 = _prep_ws(l1_har_base_w, l1_har_spline_w)
    sb1, ss1 = _prep_ws(l1_sol_base_w, l1_sol_spline_w)
    ib1, iw1 = _prep_ws(l1_irr_base_w, l1_irr_spline_w)

    degu, degd = _sc_deg(cu, cd, zerosW, onesW)
    zh0, ks0, ki0 = _tc_layer0(x, degu, degd, (hb0, hs0, sb0, ss0, ib0, iw0))
    au0, ad0 = _sc_prop(ks0, ki0, ru, cu, rd, cd, zerosD)
    zh1, ks1, ki1 = _tc_layer1(zh0, au0, ad0, degu, degd,
                               (hb1, hs1, sb1, ss1, ib1, iw1))
    au1, ad1 = _sc_prop(ks1, ki1, ru, cu, rd, cd, zerosD)
    return _tc_final(zh1, au1, ad1, degu, degd)


# speedup vs baseline: 1.1529x; 1.1529x over previous
import numpy as np
import jax
import jax.numpy as jnp
from jax import lax
from jax.experimental import pallas as pl
from jax.experimental.pallas import tpu as pltpu
from jax.experimental.pallas import tpu_sc as plsc

N = 10000
E = 320000
D = 128
GRID_SIZE = 5
SPLINE_ORDER = 3
COEF = GRID_SIZE + SPLINE_ORDER

NSUB = 16
EPW = E // NSUB
CHUNK = 128
NFULL = EPW // CHUNK
TAIL = EPW - NFULL * CHUNK
SUP_C = 12
SUP_P = SUP_C // 2
SUP_E = SUP_C * CHUNK
RPS = 624
RPS_LAST = N - 15 * RPS

ROWB = 1000
NBLK = N // ROWB

_G = np.arange(-SPLINE_ORDER, GRID_SIZE + SPLINE_ORDER + 1,
               dtype=np.float32) * np.float32(2.0 / GRID_SIZE) - np.float32(1.0)


_BSCALE = float(1.0 / (6.0 * (np.float64(_G[1]) - np.float64(_G[0])) ** 3))


def _bsplines(x):
    g = _G
    s = [x - g[j] for j in range(len(g))]
    bs = [jnp.logical_and(x >= g[j], x < g[j + 1]).astype(x.dtype)
          for j in range(len(g) - 1)]
    for k in range(1, SPLINE_ORDER + 1):
        bs = [s[j] * bs[j] - s[j + k + 1] * bs[j + 1]
              for j in range(len(bs) - 1)]
    return bs


def _dot(a, b):
    return lax.dot_general(a, b, (((1,), (0,)), ((), ())),
                           precision=lax.Precision.DEFAULT,
                           preferred_element_type=jnp.float32)


def _dinv(deg_ref):
    d = deg_ref[...][:, 0:1]
    return jnp.where(d > 0.0, lax.rsqrt(d), 0.0)


def _kan3(h, hb, hs, sb, ss, ib, iw):
    sil = jax.nn.silu(h)
    b = jnp.concatenate(_bsplines(h), axis=1)
    zh = _dot(sil, hb[...]) + _dot(b, hs[...])
    zs = _dot(sil, sb[...]) + _dot(b, ss[...])
    zi = _dot(sil, ib[...]) + _dot(b, iw[...])
    return zh, zs, zi


_ROWSPEC = pl.BlockSpec((ROWB, D), lambda i: (i, 0))
_DEGSPEC = pl.BlockSpec((ROWB, 16), lambda i: (i, 0))
_WSPEC = pl.BlockSpec((D, D), lambda i: (0, 0))
_SWSPEC = pl.BlockSpec((COEF * D, D), lambda i: (0, 0))
_OUT = jax.ShapeDtypeStruct((N, D), jnp.float32)


def _tc_layer0a(x, ws):
    def body(x_ref, hb, hs, sb, ss, ib, iw, zh_ref, ks_ref, ki_ref):
        zh, zs, zi = _kan3(x_ref[...], hb, hs, sb, ss, ib, iw)
        zh_ref[...] = zh
        ks_ref[...] = zs
        ki_ref[...] = zi
    return pl.pallas_call(
        body, grid=(NBLK,),
        in_specs=[_ROWSPEC, _WSPEC, _SWSPEC, _WSPEC, _SWSPEC, _WSPEC, _SWSPEC],
        out_specs=[_ROWSPEC, _ROWSPEC, _ROWSPEC],
        out_shape=[_OUT, _OUT, _OUT],
    )(x, *ws)


def _tc_scale(zs, zi, degu, degd):
    def body(zs_ref, zi_ref, du_ref, dd_ref, ks_ref, ki_ref):
        ks_ref[...] = zs_ref[...] * _dinv(du_ref)
        ki_ref[...] = zi_ref[...] * _dinv(dd_ref)
    return pl.pallas_call(
        body, grid=(NBLK,),
        in_specs=[_ROWSPEC, _ROWSPEC, _DEGSPEC, _DEGSPEC],
        out_specs=[_ROWSPEC, _ROWSPEC],
        out_shape=[_OUT, _OUT],
    )(zs, zi, degu, degd)


def _tc_layer1(zh0, au, ad, degu, degd, ws):
    def body(zh0_ref, au_ref, ad_ref, du_ref, dd_ref,
             hb, hs, sb, ss, ib, iw, zh_ref, ks_ref, ki_ref):
        dinvu = _dinv(du_ref)
        dinvd = _dinv(dd_ref)
        h = jax.nn.relu(zh0_ref[...] + dinvu * au_ref[...] + dinvd * ad_ref[...])
        zh, zs, zi = _kan3(h, hb, hs, sb, ss, ib, iw)
        zh_ref[...] = zh
        ks_ref[...] = zs * dinvu
        ki_ref[...] = zi * dinvd
    return pl.pallas_call(
        body, grid=(NBLK,),
        in_specs=[_ROWSPEC, _ROWSPEC, _ROWSPEC, _DEGSPEC, _DEGSPEC,
                  _WSPEC, _SWSPEC, _WSPEC, _SWSPEC, _WSPEC, _SWSPEC],
        out_specs=[_ROWSPEC, _ROWSPEC, _ROWSPEC],
        out_shape=[_OUT, _OUT, _OUT],
    )(zh0, au, ad, degu, degd, *ws)


def _tc_final(zh1, au, ad, degu, degd):
    def body(zh_ref, au_ref, ad_ref, du_ref, dd_ref, o_ref):
        dinvu = _dinv(du_ref)
        dinvd = _dinv(dd_ref)
        o_ref[...] = jax.nn.relu(
            zh_ref[...] + dinvu * au_ref[...] + dinvd * ad_ref[...])
    return pl.pallas_call(
        body, grid=(NBLK,),
        in_specs=[_ROWSPEC, _ROWSPEC, _ROWSPEC, _DEGSPEC, _DEGSPEC],
        out_specs=_ROWSPEC,
        out_shape=_OUT,
    )(zh1, au, ad, degu, degd)


def _sc_mesh():
    return plsc.VectorSubcoreMesh(core_axis_name="c", subcore_axis_name="s")


def _rowcopy(sid, src, dst):
    @pl.when(sid < 15)
    def _():
        off = pl.multiple_of(sid * RPS, 8)
        pltpu.sync_copy(src.at[pl.ds(off, RPS)], dst.at[pl.ds(off, RPS)])

    @pl.when(sid == 15)
    def _():
        pltpu.sync_copy(src.at[pl.ds(15 * RPS, RPS_LAST)],
                        dst.at[pl.ds(15 * RPS, RPS_LAST)])


DEGW = 16


def _sc_deg(cu, cd, zerosW, onesW):
    def body(cu_hbm, cd_hbm, z_hbm, o_hbm, du_hbm, dd_hbm,
             acc, cidx, cidxT, ones_v, call_v):
        cid = lax.axis_index("c")
        sid = lax.axis_index("s")
        _rowcopy(sid, z_hbm, acc)
        pltpu.sync_copy(o_hbm, ones_v)
        plsc.subcore_barrier()
        base = sid * EPW

        def run(col_hbm):
            pltpu.sync_copy(col_hbm.at[pl.ds(base, EPW)], call_v)

            def step(j, carry):
                for i in range(CHUNK // 16):
                    cidx[pl.ds(16 * i, 16)] = call_v[pl.ds(j * CHUNK + 16 * i, 16)]
                pltpu.sync_copy(ones_v, acc.at[cidx], add=True)
                return carry
            lax.fori_loop(0, NFULL, step, 0)
            for i in range(TAIL // 16):
                cidxT[pl.ds(16 * i, 16)] = call_v[pl.ds(NFULL * CHUNK + 16 * i, 16)]
            pltpu.sync_copy(ones_v.at[pl.ds(0, TAIL)], acc.at[cidxT], add=True)

        @pl.when(cid == 0)
        def _():
            run(cu_hbm)

        @pl.when(cid == 1)
        def _():
            run(cd_hbm)

        plsc.subcore_barrier()

        @pl.when(cid == 0)
        def _():
            _rowcopy(sid, acc, du_hbm)

        @pl.when(cid == 1)
        def _():
            _rowcopy(sid, acc, dd_hbm)

    f = pl.kernel(
        body,
        out_type=(jax.ShapeDtypeStruct((N, DEGW), jnp.float32),
                  jax.ShapeDtypeStruct((N, DEGW), jnp.float32)),
        mesh=_sc_mesh(),
        compiler_params=pltpu.CompilerParams(use_tc_tiling_on_sc=False),
        scratch_types=(
            pltpu.VMEM_SHARED((N, DEGW), jnp.float32),
            pltpu.VMEM((CHUNK,), jnp.int32),
            pltpu.VMEM((TAIL,), jnp.int32),
            pltpu.VMEM((CHUNK, DEGW), jnp.float32),
            pltpu.VMEM((EPW,), jnp.int32),
        ),
    )
    return f(cu, cd, zerosW, onesW)


def _sc_prop(ks, ki, ru, cu, rd, cd, zerosD):
    def body(ks_hbm, ki_hbm, ru_hbm, cu_hbm, rd_hbm, cd_hbm, z_hbm,
             au_hbm, ad_hbm, acc, ridx0, ridx1, cidx0, cidx1, ridxT, cidxT,
             rows0, rows1, rowsT, rall_v, call_v, sem0, sem1):
        cid = lax.axis_index("c")
        sid = lax.axis_index("s")
        _rowcopy(sid, z_hbm, acc)
        plsc.subcore_barrier()
        base = sid * EPW

        def run(tab_hbm, row_hbm, col_hbm):
            def stage(s):
                pltpu.sync_copy(row_hbm.at[pl.ds(base + s * SUP_E, SUP_E)], rall_v)
                pltpu.sync_copy(col_hbm.at[pl.ds(base + s * SUP_E, SUP_E)], call_v)

            def load_idx(loc, ridx, cidx):
                for i in range(CHUNK // 16):
                    ridx[pl.ds(16 * i, 16)] = rall_v[pl.ds(loc + 16 * i, 16)]
                    cidx[pl.ds(16 * i, 16)] = call_v[pl.ds(loc + 16 * i, 16)]

            stage(0)
            load_idx(0, ridx0, cidx0)
            pltpu.async_copy(tab_hbm.at[ridx0], rows0, sem0)

            def pair(t, carry):
                j1loc = (2 * t + 1) * CHUNK - (t // SUP_P) * SUP_E
                load_idx(j1loc, ridx1, cidx1)
                pltpu.async_copy(tab_hbm.at[ridx1], rows1, sem1)
                pltpu.make_async_copy(tab_hbm.at[ridx0], rows0, sem0).wait()
                pltpu.sync_copy(rows0, acc.at[cidx0], add=True)

                @pl.when(jnp.logical_and(t % SUP_P == SUP_P - 1,
                                         t < NFULL // 2 - 1))
                def _():
                    stage(t // SUP_P + 1)

                @pl.when(t < NFULL // 2 - 1)
                def _():
                    j0loc = (2 * t + 2) * CHUNK - ((2 * t + 2) // SUP_C) * SUP_E
                    load_idx(j0loc, ridx0, cidx0)
                    pltpu.async_copy(tab_hbm.at[ridx0], rows0, sem0)
                pltpu.make_async_copy(tab_hbm.at[ridx1], rows1, sem1).wait()
                pltpu.sync_copy(rows1, acc.at[cidx1], add=True)
                return carry
            lax.fori_loop(0, NFULL // 2, pair, 0)

            pltpu.sync_copy(row_hbm.at[pl.ds(base + NFULL * CHUNK, TAIL)], ridxT)
            pltpu.sync_copy(col_hbm.at[pl.ds(base + NFULL * CHUNK, TAIL)], cidxT)
            pltpu.async_copy(tab_hbm.at[ridxT], rowsT, sem0).wait()
            pltpu.sync_copy(rowsT, acc.at[cidxT], add=True)

        @pl.when(cid == 0)
        def _():
            run(ks_hbm, ru_hbm, cu_hbm)

        @pl.when(cid == 1)
        def _():
            run(ki_hbm, rd_hbm, cd_hbm)

        plsc.subcore_barrier()

        @pl.when(cid == 0)
        def _():
            _rowcopy(sid, acc, au_hbm)

        @pl.when(cid == 1)
        def _():
            _rowcopy(sid, acc, ad_hbm)

    f = pl.kernel(
        body,
        out_type=(jax.ShapeDtypeStruct((N, D), jnp.float32),
                  jax.ShapeDtypeStruct((N, D), jnp.float32)),
        mesh=_sc_mesh(),
        scratch_types=(
            pltpu.VMEM_SHARED((N, D), jnp.float32),
            pltpu.VMEM((CHUNK,), jnp.int32),
            pltpu.VMEM((CHUNK,), jnp.int32),
            pltpu.VMEM((CHUNK,), jnp.int32),
            pltpu.VMEM((CHUNK,), jnp.int32),
            pltpu.VMEM((TAIL,), jnp.int32),
            pltpu.VMEM((TAIL,), jnp.int32),
            pltpu.VMEM((CHUNK, D), jnp.float32),
            pltpu.VMEM((CHUNK, D), jnp.float32),
            pltpu.VMEM((TAIL, D), jnp.float32),
            pltpu.VMEM((SUP_E,), jnp.int32),
            pltpu.VMEM((SUP_E,), jnp.int32),
            pltpu.SemaphoreType.DMA,
            pltpu.SemaphoreType.DMA,
        ),
    )
    return f(ks, ki, ru, cu, rd, cd, zerosD)


def _prep_ws(bw, sw):
    sw_t = jnp.transpose(sw, (2, 1, 0)).reshape(COEF * D, D)
    return bw.T, sw_t * jnp.float32(_BSCALE)


def kernel(x, Ld, Lu, l0_har_base_w, l0_har_spline_w, l0_sol_base_w,
           l0_sol_spline_w, l0_irr_base_w, l0_irr_spline_w, l1_har_base_w,
           l1_har_spline_w, l1_sol_base_w, l1_sol_spline_w, l1_irr_base_w,
           l1_irr_spline_w):
    ru, cu = Lu[0], Lu[1]
    rd, cd = Ld[0], Ld[1]
    zerosD = jnp.zeros((N, D), jnp.float32)
    zerosW = jnp.zeros((N, DEGW), jnp.float32)
    onesW = jnp.ones((CHUNK, DEGW), jnp.float32)

    hb0, hs0 = _prep_ws(l0_har_base_w, l0_har_spline_w)
    sb0, ss0 = _prep_ws(l0_sol_base_w, l0_sol_spline_w)
    ib0, iw0 = _prep_ws(l0_irr_base_w, l0_irr_spline_w)
    hb1, hs1 = _prep_ws(l1_har_base_w, l1_har_spline_w)
    sb1, ss1 = _prep_ws(l1_sol_base_w, l1_sol_spline_w)
    ib1, iw1 = _prep_ws(l1_irr_base_w, l1_irr_spline_w)

    degu, degd = _sc_deg(cu, cd, zerosW, onesW)
    zh0, zs0, zi0 = _tc_layer0a(x, (hb0, hs0, sb0, ss0, ib0, iw0))
    ks0, ki0 = _tc_scale(zs0, zi0, degu, degd)
    au0, ad0 = _sc_prop(ks0, ki0, ru, cu, rd, cd, zerosD)
    zh1, ks1, ki1 = _tc_layer1(zh0, au0, ad0, degu, degd,
                               (hb1, hs1, sb1, ss1, ib1, iw1))
    au1, ad1 = _sc_prop(ks1, ki1, ru, cu, rd, cd, zerosD)
    return _tc_final(zh1, au1, ad1, degu, degd)


# speedup vs baseline: 1.1709x; 1.0156x over previous
import numpy as np
import jax
import jax.numpy as jnp
from jax import lax
from jax.experimental import pallas as pl
from jax.experimental.pallas import tpu as pltpu
from jax.experimental.pallas import tpu_sc as plsc

N = 10000
E = 320000
D = 128
GRID_SIZE = 5
SPLINE_ORDER = 3
COEF = GRID_SIZE + SPLINE_ORDER

NSUB = 16
EPW = E // NSUB
CHUNK = 128
NFULL = EPW // CHUNK
TAIL = EPW - NFULL * CHUNK
SUP_C = 12
SUP_P = SUP_C // 2
SUP_E = SUP_C * CHUNK
RPS = 624
RPS_LAST = N - 15 * RPS

ROWB = 2000
NBLK = N // ROWB

_G = np.arange(-SPLINE_ORDER, GRID_SIZE + SPLINE_ORDER + 1,
               dtype=np.float32) * np.float32(2.0 / GRID_SIZE) - np.float32(1.0)


_BSCALE = float(1.0 / (6.0 * (np.float64(_G[1]) - np.float64(_G[0])) ** 3))


def _bsplines(x):
    g = _G
    s = [x - g[j] for j in range(len(g))]
    bs = [jnp.logical_and(x >= g[j], x < g[j + 1]).astype(x.dtype)
          for j in range(len(g) - 1)]
    for k in range(1, SPLINE_ORDER + 1):
        bs = [s[j] * bs[j] - s[j + k + 1] * bs[j + 1]
              for j in range(len(bs) - 1)]
    return bs


def _dot(a, b):
    return lax.dot_general(a, b, (((1,), (0,)), ((), ())),
                           precision=lax.Precision.DEFAULT,
                           preferred_element_type=jnp.float32)


def _dinv(deg_ref):
    d = deg_ref[...][:, 0:1]
    return jnp.where(d > 0.0, lax.rsqrt(d), 0.0)


def _kan3(h, hb, hs, sb, ss, ib, iw):
    sil = jax.nn.silu(h)
    b = jnp.concatenate(_bsplines(h), axis=1)
    zh = _dot(sil, hb[...]) + _dot(b, hs[...])
    zs = _dot(sil, sb[...]) + _dot(b, ss[...])
    zi = _dot(sil, ib[...]) + _dot(b, iw[...])
    return zh, zs, zi


_ROWSPEC = pl.BlockSpec((ROWB, D), lambda i: (i, 0))
_DEGSPEC = pl.BlockSpec((ROWB, 16), lambda i: (i, 0))
_WSPEC = pl.BlockSpec((D, D), lambda i: (0, 0))
_SWSPEC = pl.BlockSpec((COEF * D, D), lambda i: (0, 0))
_OUT = jax.ShapeDtypeStruct((N, D), jnp.float32)


def _tc_layer0a(x, ws):
    def body(x_ref, hb, hs, sb, ss, ib, iw, zh_ref, ks_ref, ki_ref):
        zh, zs, zi = _kan3(x_ref[...], hb, hs, sb, ss, ib, iw)
        zh_ref[...] = zh
        ks_ref[...] = zs
        ki_ref[...] = zi
    return pl.pallas_call(
        body, grid=(NBLK,),
        in_specs=[_ROWSPEC, _WSPEC, _SWSPEC, _WSPEC, _SWSPEC, _WSPEC, _SWSPEC],
        out_specs=[_ROWSPEC, _ROWSPEC, _ROWSPEC],
        out_shape=[_OUT, _OUT, _OUT],
    )(x, *ws)


def _tc_scale(zs, zi, degu, degd):
    def body(zs_ref, zi_ref, du_ref, dd_ref, ks_ref, ki_ref):
        ks_ref[...] = zs_ref[...] * _dinv(du_ref)
        ki_ref[...] = zi_ref[...] * _dinv(dd_ref)
    return pl.pallas_call(
        body, grid=(NBLK,),
        in_specs=[_ROWSPEC, _ROWSPEC, _DEGSPEC, _DEGSPEC],
        out_specs=[_ROWSPEC, _ROWSPEC],
        out_shape=[_OUT, _OUT],
    )(zs, zi, degu, degd)


def _tc_layer1(zh0, au, ad, degu, degd, ws):
    def body(zh0_ref, au_ref, ad_ref, du_ref, dd_ref,
             hb, hs, sb, ss, ib, iw, zh_ref, ks_ref, ki_ref):
        dinvu = _dinv(du_ref)
        dinvd = _dinv(dd_ref)
        h = jax.nn.relu(zh0_ref[...] + dinvu * au_ref[...] + dinvd * ad_ref[...])
        zh, zs, zi = _kan3(h, hb, hs, sb, ss, ib, iw)
        zh_ref[...] = zh
        ks_ref[...] = zs * dinvu
        ki_ref[...] = zi * dinvd
    return pl.pallas_call(
        body, grid=(NBLK,),
        in_specs=[_ROWSPEC, _ROWSPEC, _ROWSPEC, _DEGSPEC, _DEGSPEC,
                  _WSPEC, _SWSPEC, _WSPEC, _SWSPEC, _WSPEC, _SWSPEC],
        out_specs=[_ROWSPEC, _ROWSPEC, _ROWSPEC],
        out_shape=[_OUT, _OUT, _OUT],
    )(zh0, au, ad, degu, degd, *ws)


def _tc_final(zh1, au, ad, degu, degd):
    def body(zh_ref, au_ref, ad_ref, du_ref, dd_ref, o_ref):
        dinvu = _dinv(du_ref)
        dinvd = _dinv(dd_ref)
        o_ref[...] = jax.nn.relu(
            zh_ref[...] + dinvu * au_ref[...] + dinvd * ad_ref[...])
    return pl.pallas_call(
        body, grid=(NBLK,),
        in_specs=[_ROWSPEC, _ROWSPEC, _ROWSPEC, _DEGSPEC, _DEGSPEC],
        out_specs=_ROWSPEC,
        out_shape=_OUT,
    )(zh1, au, ad, degu, degd)


def _sc_mesh():
    return plsc.VectorSubcoreMesh(core_axis_name="c", subcore_axis_name="s")


def _rowcopy(sid, src, dst):
    @pl.when(sid < 15)
    def _():
        off = pl.multiple_of(sid * RPS, 8)
        pltpu.sync_copy(src.at[pl.ds(off, RPS)], dst.at[pl.ds(off, RPS)])

    @pl.when(sid == 15)
    def _():
        pltpu.sync_copy(src.at[pl.ds(15 * RPS, RPS_LAST)],
                        dst.at[pl.ds(15 * RPS, RPS_LAST)])


DEGW = 16


def _sc_deg(cu, cd, zerosW, onesW):
    def body(cu_hbm, cd_hbm, z_hbm, o_hbm, du_hbm, dd_hbm,
             acc, cidx, cidxT, ones_v, call_v):
        cid = lax.axis_index("c")
        sid = lax.axis_index("s")
        _rowcopy(sid, z_hbm, acc)
        pltpu.sync_copy(o_hbm, ones_v)
        plsc.subcore_barrier()
        base = sid * EPW

        def run(col_hbm):
            pltpu.sync_copy(col_hbm.at[pl.ds(base, EPW)], call_v)

            def step(j, carry):
                for i in range(CHUNK // 16):
                    cidx[pl.ds(16 * i, 16)] = call_v[pl.ds(j * CHUNK + 16 * i, 16)]
                pltpu.sync_copy(ones_v, acc.at[cidx], add=True)
                return carry
            lax.fori_loop(0, NFULL, step, 0)
            for i in range(TAIL // 16):
                cidxT[pl.ds(16 * i, 16)] = call_v[pl.ds(NFULL * CHUNK + 16 * i, 16)]
            pltpu.sync_copy(ones_v.at[pl.ds(0, TAIL)], acc.at[cidxT], add=True)

        @pl.when(cid == 0)
        def _():
            run(cu_hbm)

        @pl.when(cid == 1)
        def _():
            run(cd_hbm)

        plsc.subcore_barrier()

        @pl.when(cid == 0)
        def _():
            _rowcopy(sid, acc, du_hbm)

        @pl.when(cid == 1)
        def _():
            _rowcopy(sid, acc, dd_hbm)

    f = pl.kernel(
        body,
        out_type=(jax.ShapeDtypeStruct((N, DEGW), jnp.float32),
                  jax.ShapeDtypeStruct((N, DEGW), jnp.float32)),
        mesh=_sc_mesh(),
        compiler_params=pltpu.CompilerParams(use_tc_tiling_on_sc=False),
        scratch_types=(
            pltpu.VMEM_SHARED((N, DEGW), jnp.float32),
            pltpu.VMEM((CHUNK,), jnp.int32),
            pltpu.VMEM((TAIL,), jnp.int32),
            pltpu.VMEM((CHUNK, DEGW), jnp.float32),
            pltpu.VMEM((EPW,), jnp.int32),
        ),
    )
    return f(cu, cd, zerosW, onesW)


def _sc_prop(ks, ki, ru, cu, rd, cd, zerosD):
    def body(ks_hbm, ki_hbm, ru_hbm, cu_hbm, rd_hbm, cd_hbm, z_hbm,
             au_hbm, ad_hbm, acc, ridx0, ridx1, cidx0, cidx1, ridxT, cidxT,
             rows0, rows1, rowsT, rall_v, call_v, sem0, sem1):
        cid = lax.axis_index("c")
        sid = lax.axis_index("s")
        _rowcopy(sid, z_hbm, acc)
        plsc.subcore_barrier()
        base = sid * EPW

        def run(tab_hbm, row_hbm, col_hbm):
            def stage(s):
                pltpu.sync_copy(row_hbm.at[pl.ds(base + s * SUP_E, SUP_E)], rall_v)
                pltpu.sync_copy(col_hbm.at[pl.ds(base + s * SUP_E, SUP_E)], call_v)

            def load_idx(loc, ridx, cidx):
                for i in range(CHUNK // 16):
                    ridx[pl.ds(16 * i, 16)] = rall_v[pl.ds(loc + 16 * i, 16)]
                    cidx[pl.ds(16 * i, 16)] = call_v[pl.ds(loc + 16 * i, 16)]

            stage(0)
            load_idx(0, ridx0, cidx0)
            pltpu.async_copy(tab_hbm.at[ridx0], rows0, sem0)

            def pair(t, carry):
                j1loc = (2 * t + 1) * CHUNK - (t // SUP_P) * SUP_E
                load_idx(j1loc, ridx1, cidx1)
                pltpu.async_copy(tab_hbm.at[ridx1], rows1, sem1)
                pltpu.make_async_copy(tab_hbm.at[ridx0], rows0, sem0).wait()
                pltpu.sync_copy(rows0, acc.at[cidx0], add=True)

                @pl.when(jnp.logical_and(t % SUP_P == SUP_P - 1,
                                         t < NFULL // 2 - 1))
                def _():
                    stage(t // SUP_P + 1)

                @pl.when(t < NFULL // 2 - 1)
                def _():
                    j0loc = (2 * t + 2) * CHUNK - ((2 * t + 2) // SUP_C) * SUP_E
                    load_idx(j0loc, ridx0, cidx0)
                    pltpu.async_copy(tab_hbm.at[ridx0], rows0, sem0)
                pltpu.make_async_copy(tab_hbm.at[ridx1], rows1, sem1).wait()
                pltpu.sync_copy(rows1, acc.at[cidx1], add=True)
                return carry
            lax.fori_loop(0, NFULL // 2, pair, 0)

            pltpu.sync_copy(row_hbm.at[pl.ds(base + NFULL * CHUNK, TAIL)], ridxT)
            pltpu.sync_copy(col_hbm.at[pl.ds(base + NFULL * CHUNK, TAIL)], cidxT)
            pltpu.async_copy(tab_hbm.at[ridxT], rowsT, sem0).wait()
            pltpu.sync_copy(rowsT, acc.at[cidxT], add=True)

        @pl.when(cid == 0)
        def _():
            run(ks_hbm, ru_hbm, cu_hbm)

        @pl.when(cid == 1)
        def _():
            run(ki_hbm, rd_hbm, cd_hbm)

        plsc.subcore_barrier()

        @pl.when(cid == 0)
        def _():
            _rowcopy(sid, acc, au_hbm)

        @pl.when(cid == 1)
        def _():
            _rowcopy(sid, acc, ad_hbm)

    f = pl.kernel(
        body,
        out_type=(jax.ShapeDtypeStruct((N, D), jnp.float32),
                  jax.ShapeDtypeStruct((N, D), jnp.float32)),
        mesh=_sc_mesh(),
        scratch_types=(
            pltpu.VMEM_SHARED((N, D), jnp.float32),
            pltpu.VMEM((CHUNK,), jnp.int32),
            pltpu.VMEM((CHUNK,), jnp.int32),
            pltpu.VMEM((CHUNK,), jnp.int32),
            pltpu.VMEM((CHUNK,), jnp.int32),
            pltpu.VMEM((TAIL,), jnp.int32),
            pltpu.VMEM((TAIL,), jnp.int32),
            pltpu.VMEM((CHUNK, D), jnp.float32),
            pltpu.VMEM((CHUNK, D), jnp.float32),
            pltpu.VMEM((TAIL, D), jnp.float32),
            pltpu.VMEM((SUP_E,), jnp.int32),
            pltpu.VMEM((SUP_E,), jnp.int32),
            pltpu.SemaphoreType.DMA,
            pltpu.SemaphoreType.DMA,
        ),
    )
    return f(ks, ki, ru, cu, rd, cd, zerosD)


def _prep_ws(bw, sw):
    sw_t = jnp.transpose(sw, (2, 1, 0)).reshape(COEF * D, D)
    return bw.T, sw_t * jnp.float32(_BSCALE)


def kernel(x, Ld, Lu, l0_har_base_w, l0_har_spline_w, l0_sol_base_w,
           l0_sol_spline_w, l0_irr_base_w, l0_irr_spline_w, l1_har_base_w,
           l1_har_spline_w, l1_sol_base_w, l1_sol_spline_w, l1_irr_base_w,
           l1_irr_spline_w):
    ru, cu = Lu[0], Lu[1]
    rd, cd = Ld[0], Ld[1]
    zerosD = jnp.zeros((N, D), jnp.float32)
    zerosW = jnp.zeros((N, DEGW), jnp.float32)
    onesW = jnp.ones((CHUNK, DEGW), jnp.float32)

    hb0, hs0 = _prep_ws(l0_har_base_w, l0_har_spline_w)
    sb0, ss0 = _prep_ws(l0_sol_base_w, l0_sol_spline_w)
    ib0, iw0 = _prep_ws(l0_irr_base_w, l0_irr_spline_w)
    hb1, hs1 = _prep_ws(l1_har_base_w, l1_har_spline_w)
    sb1, ss1 = _prep_ws(l1_sol_base_w, l1_sol_spline_w)
    ib1, iw1 = _prep_ws(l1_irr_base_w, l1_irr_spline_w)

    degu, degd = _sc_deg(cu, cd, zerosW, onesW)
    zh0, zs0, zi0 = _tc_layer0a(x, (hb0, hs0, sb0, ss0, ib0, iw0))
    ks0, ki0 = _tc_scale(zs0, zi0, degu, degd)
    au0, ad0 = _sc_prop(ks0, ki0, ru, cu, rd, cd, zerosD)
    zh1, ks1, ki1 = _tc_layer1(zh0, au0, ad0, degu, degd,
                               (hb1, hs1, sb1, ss1, ib1, iw1))
    au1, ad1 = _sc_prop(ks1, ki1, ru, cu, rd, cd, zerosD)
    return _tc_final(zh1, au1, ad1, degu, degd)
